# baseline (speedup 1.0000x reference)
import functools

import jax
import jax.numpy as jnp
from jax import lax
from jax.experimental import pallas as pl
from jax.experimental.pallas import tpu as pltpu
from jax.experimental.pallas import tpu_sc as plsc

D = 64
B = 4096
L = 200
NC = 2
NS = 16
NW = NC * NS
BPW = B // NW
LI = L // 8
SCALE = 8.0

_mesh = plsc.VectorSubcoreMesh(core_axis_name="c", subcore_axis_name="s")


@functools.partial(
    pl.kernel,
    out_type=jax.ShapeDtypeStruct((L, D // 8, NW, 8, 128), jnp.float32),
    mesh=_mesh,
    scratch_types=[
        pltpu.VMEM((LI, 8, 128), jnp.int32),
        pltpu.VMEM((1, 128), jnp.int32),
        pltpu.VMEM((2, 128), jnp.int32),
        pltpu.VMEM((2, 128), jnp.int32),
        pltpu.VMEM((2 * 128, 128), jnp.float32),
        pltpu.VMEM((2 * 8, 8, 128), jnp.float32),
        pltpu.SemaphoreType.DMA,
        pltpu.SemaphoreType.DMA,
        pltpu.SemaphoreType.DMA,
        pltpu.SemaphoreType.DMA,
        pltpu.SemaphoreType.DMA,
    ],
    compiler_params=pltpu.CompilerParams(
        use_tc_tiling_on_sc=True, needs_layout_passes=False),
)
def _embed(uni_v, task_v, ttab2, tab2, out5,
           idx_all, tidx_v, idx2_v, half_v, g_v, t_v,
           ss, sg0, sg1, sw0, sw1):
    sg = (sg0, sg1)
    sw = (sw0, sw1)
    wid = lax.axis_index("s") * NC + lax.axis_index("c")
    iota = lax.iota(jnp.int32, 16)

    def prep_from(src_row, cb):
        for s in range(8):
            v = src_row(s)
            idx2_v[cb, pl.ds(s * 16, 16)] = ((v >> 13) << 12) + (v & 4095)
            half_v[cb, pl.ds(s * 16, 16)] = ((v >> 12) & 1) * 64

    def prep_task(cb):
        for s in range(8):
            v = tidx_v[0, pl.ds(s * 16, 16)]
            ge = (v >= 500).astype(jnp.int32)
            idx2_v[cb, pl.ds(s * 16, 16)] = v - ge * 500
            half_v[cb, pl.ds(s * 16, 16)] = ge * 64

    def prep(l, cb):
        prep_from(lambda s: idx_all[l // 8, l % 8, pl.ds(s * 16, 16)], cb)

    def fire_gather(tab, cb):
        pltpu.async_copy(tab.at[idx2_v.at[cb]],
                         g_v.at[pl.ds(cb * 128, 128)], sg[cb])

    def drain_gather(cb):
        pltpu.make_async_copy(tab2.at[pl.ds(0, 128)],
                              g_v.at[pl.ds(cb * 128, 128)], sg[cb]).wait()

    def fire_write(l, cb):
        pltpu.async_copy(t_v.at[pl.ds(cb * 8, 8)],
                         out5.at[l, :, wid], sw[cb])

    def drain_write(cb):
        pltpu.make_async_copy(t_v.at[pl.ds(cb * 8, 8)],
                              out5.at[0, :, 0], sw[cb]).wait()

    def extract(cb):
        for bc0 in range(0, 128, 16):
            rows = iota + (cb * 128 + bc0)
            pc = half_v[cb, pl.ds(bc0, 16)]

            for d in range(D):
                vals = plsc.load_gather(g_v, [rows, pc + d])
                t_v[cb * 8 + d // 8, d % 8, pl.ds(bc0, 16)] = vals

    for li in range(LI):
        pltpu.async_copy(uni_v.at[li, wid], idx_all.at[li], ss)
    pltpu.async_copy(task_v.at[0, wid, 0], tidx_v.at[0], ss)
    for li in range(LI):
        pltpu.make_async_copy(uni_v.at[li, 0], idx_all.at[li], ss).wait()
    pltpu.make_async_copy(task_v.at[0, 0, 0], tidx_v.at[0], ss).wait()

    prep_task(0)
    fire_gather(ttab2, 0)
    prep(1, 1)
    fire_gather(tab2, 1)

    @pl.loop(0, L, step=2)
    def _(g):
        for cb in range(2):
            l = g + cb
            drain_gather(cb)

            @pl.when(g >= 2)
            def _():
                drain_write(cb)

            extract(cb)
            fire_write(l, cb)

            @pl.when(l + 2 < L)
            def _():
                prep(l + 2, cb)
                fire_gather(tab2, cb)

    drain_write(0)
    drain_write(1)


def _repack_body(x_ref, o_ref):
    rb = o_ref.shape[0]
    x = x_ref[...]
    o_ref[:, 0:D] = x[:, :rb].T * SCALE
    o_ref[:, D:128] = x[:, rb:].T * SCALE


def _repack(tab_t, rb):
    n = tab_t.shape[1]
    nb = (n + 2 * rb - 1) // (2 * rb)
    return pl.pallas_call(
        _repack_body,
        grid=(nb,),
        in_specs=[pl.BlockSpec((D, 2 * rb), lambda i: (0, i))],
        out_specs=pl.BlockSpec((rb, 128), lambda i: (i, 0)),
        out_shape=jax.ShapeDtypeStruct((nb * rb, 128), jnp.float32),
    )(tab_t)


def kernel(task, uni, task_table, uni_table):
    uni_v = uni.T.reshape(LI, 8, NW, 128).transpose(0, 2, 1, 3)
    task_v = task.T.reshape(LI, 8, NW, 128).transpose(0, 2, 1, 3)
    tab2 = _repack(uni_table.T, 4096)
    ttab2 = _repack(task_table.T, 500)
    out5 = _embed(uni_v, task_v, ttab2, tab2)
    return out5.transpose(2, 4, 0, 1, 3).reshape(B, L, D)


# speedup vs baseline: 1.6107x; 1.6107x over previous
import functools

import jax
import jax.numpy as jnp
from jax import lax
from jax.experimental import pallas as pl
from jax.experimental.pallas import tpu as pltpu
from jax.experimental.pallas import tpu_sc as plsc

D = 64
B = 4096
L = 200
NC = 2
NS = 16
NW = NC * NS
BPW = B // NW
LI = L // 8
SCALE = 8.0

_mesh = plsc.VectorSubcoreMesh(core_axis_name="c", subcore_axis_name="s")


@functools.partial(
    pl.kernel,
    out_type=jax.ShapeDtypeStruct((L, D // 8, NW, 8, 128), jnp.float32),
    mesh=_mesh,
    scratch_types=[
        pltpu.VMEM((LI, 8, 128), jnp.int32),
        pltpu.VMEM((1, 128), jnp.int32),
        pltpu.VMEM((2, 128), jnp.int32),
        pltpu.VMEM((2, 128), jnp.int32),
        pltpu.VMEM((2 * 128, 128), jnp.float32),
        pltpu.VMEM((2 * 8, 8, 128), jnp.float32),
        pltpu.SemaphoreType.DMA,
        pltpu.SemaphoreType.DMA,
        pltpu.SemaphoreType.DMA,
        pltpu.SemaphoreType.DMA,
        pltpu.SemaphoreType.DMA,
    ],
    compiler_params=pltpu.CompilerParams(
        use_tc_tiling_on_sc=True, needs_layout_passes=False),
)
def _embed(uni_v, task_v, ttab2, tab2, out5,
           idx_all, tidx_v, idx2_v, half_v, g_v, t_v,
           ss, sg0, sg1, sw0, sw1):
    sg = (sg0, sg1)
    sw = (sw0, sw1)
    wid = lax.axis_index("s") * NC + lax.axis_index("c")
    iota = lax.iota(jnp.int32, 16)

    def prep_from(src_row, cb):
        for s in range(8):
            v = src_row(s)
            idx2_v[cb, pl.ds(s * 16, 16)] = ((v >> 13) << 12) + (v & 4095)
            half_v[cb, pl.ds(s * 16, 16)] = ((v >> 12) & 1) * 64

    def prep_task(cb):
        for s in range(8):
            v = tidx_v[0, pl.ds(s * 16, 16)]
            ge = (v >= 500).astype(jnp.int32)
            idx2_v[cb, pl.ds(s * 16, 16)] = v - ge * 500
            half_v[cb, pl.ds(s * 16, 16)] = ge * 64

    def prep(l, cb):
        prep_from(lambda s: idx_all[l // 8, l % 8, pl.ds(s * 16, 16)], cb)

    def fire_gather(tab, cb):
        pltpu.async_copy(tab.at[idx2_v.at[cb]],
                         g_v.at[pl.ds(cb * 128, 128)], sg[cb])

    def drain_gather(cb):
        pltpu.make_async_copy(tab2.at[pl.ds(0, 128)],
                              g_v.at[pl.ds(cb * 128, 128)], sg[cb]).wait()

    def fire_write(l, cb):
        pltpu.async_copy(t_v.at[pl.ds(cb * 8, 8)],
                         out5.at[l, :, wid], sw[cb])

    def drain_write(cb):
        pltpu.make_async_copy(t_v.at[pl.ds(cb * 8, 8)],
                              out5.at[0, :, 0], sw[cb]).wait()

    def extract(cb):
        for bc0 in range(0, 128, 16):
            rows = iota + (cb * 128 + bc0)
            pc = half_v[cb, pl.ds(bc0, 16)]

            @plsc.parallel_loop(0, D, unroll=8)
            def _(d):
                vals = plsc.load_gather(g_v, [rows, pc + d])
                t_v[cb * 8 + d // 8, d % 8, pl.ds(bc0, 16)] = vals

    for li in range(LI):
        pltpu.async_copy(uni_v.at[li, wid], idx_all.at[li], ss)
    pltpu.async_copy(task_v.at[0, wid, 0], tidx_v.at[0], ss)
    for li in range(LI):
        pltpu.make_async_copy(uni_v.at[li, 0], idx_all.at[li], ss).wait()
    pltpu.make_async_copy(task_v.at[0, 0, 0], tidx_v.at[0], ss).wait()

    prep_task(0)
    fire_gather(ttab2, 0)
    prep(1, 1)
    fire_gather(tab2, 1)

    @pl.loop(0, L, step=2)
    def _(g):
        for cb in range(2):
            l = g + cb
            drain_gather(cb)

            @pl.when(g >= 2)
            def _():
                drain_write(cb)

            extract(cb)
            fire_write(l, cb)

            @pl.when(l + 2 < L)
            def _():
                prep(l + 2, cb)
                fire_gather(tab2, cb)

    drain_write(0)
    drain_write(1)


def _repack_body(x_ref, o_ref):
    rb = o_ref.shape[0]
    x = x_ref[...]
    o_ref[:, 0:D] = x[:, :rb].T * SCALE
    o_ref[:, D:128] = x[:, rb:].T * SCALE


def _repack(tab_t, rb):
    n = tab_t.shape[1]
    nb = (n + 2 * rb - 1) // (2 * rb)
    return pl.pallas_call(
        _repack_body,
        grid=(nb,),
        in_specs=[pl.BlockSpec((D, 2 * rb), lambda i: (0, i))],
        out_specs=pl.BlockSpec((rb, 128), lambda i: (i, 0)),
        out_shape=jax.ShapeDtypeStruct((nb * rb, 128), jnp.float32),
    )(tab_t)


def kernel(task, uni, task_table, uni_table):
    uni_v = uni.T.reshape(LI, 8, NW, 128).transpose(0, 2, 1, 3)
    task_v = task.T.reshape(LI, 8, NW, 128).transpose(0, 2, 1, 3)
    tab2 = _repack(uni_table.T, 4096)
    ttab2 = _repack(task_table.T, 500)
    out5 = _embed(uni_v, task_v, ttab2, tab2)
    return out5.transpose(2, 4, 0, 1, 3).reshape(B, L, D)


# speedup vs baseline: 1.6130x; 1.0014x over previous
import functools

import jax
import jax.numpy as jnp
from jax import lax
from jax.experimental import pallas as pl
from jax.experimental.pallas import tpu as pltpu
from jax.experimental.pallas import tpu_sc as plsc

D = 64
B = 4096
L = 200
NC = 2
NS = 16
NW = NC * NS
BPW = B // NW
LI = L // 8
SCALE = 8.0

_mesh = plsc.VectorSubcoreMesh(core_axis_name="c", subcore_axis_name="s")


@functools.partial(
    pl.kernel,
    out_type=jax.ShapeDtypeStruct((L, D // 8, NW, 8, 128), jnp.float32),
    mesh=_mesh,
    scratch_types=[
        pltpu.VMEM((LI, 8, 128), jnp.int32),
        pltpu.VMEM((1, 128), jnp.int32),
        pltpu.VMEM((2, 128), jnp.int32),
        pltpu.VMEM((2, 128), jnp.int32),
        pltpu.VMEM((2 * 128, 128), jnp.float32),
        pltpu.VMEM((2 * 8, 8, 128), jnp.float32),
        pltpu.SemaphoreType.DMA,
        pltpu.SemaphoreType.DMA,
        pltpu.SemaphoreType.DMA,
        pltpu.SemaphoreType.DMA,
        pltpu.SemaphoreType.DMA,
    ],
    compiler_params=pltpu.CompilerParams(
        use_tc_tiling_on_sc=True, needs_layout_passes=False),
)
def _embed(uni_v, task_v, ttab2, tab2, out5,
           idx_all, tidx_v, idx2_v, half_v, g_v, t_v,
           ss, sg0, sg1, sw0, sw1):
    sg = (sg0, sg1)
    sw = (sw0, sw1)
    wid = lax.axis_index("s") * NC + lax.axis_index("c")
    iota = lax.iota(jnp.int32, 16)

    def prep_from(src_row, cb):
        for s in range(8):
            v = src_row(s)
            idx2_v[cb, pl.ds(s * 16, 16)] = ((v >> 13) << 12) + (v & 4095)
            half_v[cb, pl.ds(s * 16, 16)] = ((v >> 12) & 1) * 64

    def prep_task(cb):
        for s in range(8):
            v = tidx_v[0, pl.ds(s * 16, 16)]
            ge = (v >= 500).astype(jnp.int32)
            idx2_v[cb, pl.ds(s * 16, 16)] = v - ge * 500
            half_v[cb, pl.ds(s * 16, 16)] = ge * 64

    def prep(l, cb):
        prep_from(lambda s: idx_all[l // 8, l % 8, pl.ds(s * 16, 16)], cb)

    def fire_gather(tab, cb):
        pltpu.async_copy(tab.at[idx2_v.at[cb]],
                         g_v.at[pl.ds(cb * 128, 128)], sg[cb])

    def drain_gather(cb):
        pltpu.make_async_copy(tab2.at[pl.ds(0, 128)],
                              g_v.at[pl.ds(cb * 128, 128)], sg[cb]).wait()

    def fire_write(l, cb):
        pltpu.async_copy(t_v.at[pl.ds(cb * 8, 8)],
                         out5.at[l, :, wid], sw[cb])

    def drain_write(cb):
        pltpu.make_async_copy(t_v.at[pl.ds(cb * 8, 8)],
                              out5.at[0, :, 0], sw[cb]).wait()

    def extract(cb):
        for bc0 in range(0, 128, 16):
            rows = iota + (cb * 128 + bc0)
            pc = half_v[cb, pl.ds(bc0, 16)]

            @plsc.parallel_loop(0, D, unroll=16)
            def _(d):
                vals = plsc.load_gather(g_v, [rows, pc + d])
                t_v[cb * 8 + d // 8, d % 8, pl.ds(bc0, 16)] = vals

    for li in range(LI):
        pltpu.async_copy(uni_v.at[li, wid], idx_all.at[li], ss)
    pltpu.async_copy(task_v.at[0, wid, 0], tidx_v.at[0], ss)
    for li in range(LI):
        pltpu.make_async_copy(uni_v.at[li, 0], idx_all.at[li], ss).wait()
    pltpu.make_async_copy(task_v.at[0, 0, 0], tidx_v.at[0], ss).wait()

    prep_task(0)
    fire_gather(ttab2, 0)
    prep(1, 1)
    fire_gather(tab2, 1)

    @pl.loop(0, L, step=2)
    def _(g):
        for cb in range(2):
            l = g + cb
            drain_gather(cb)

            @pl.when(g >= 2)
            def _():
                drain_write(cb)

            extract(cb)
            fire_write(l, cb)

            @pl.when(l + 2 < L)
            def _():
                prep(l + 2, cb)
                fire_gather(tab2, cb)

    drain_write(0)
    drain_write(1)


def _repack_body(x_ref, o_ref):
    rb = o_ref.shape[0]
    x = x_ref[...]
    o_ref[:, 0:D] = x[:, :rb].T * SCALE
    o_ref[:, D:128] = x[:, rb:].T * SCALE


def _repack(tab_t, rb):
    n = tab_t.shape[1]
    nb = (n + 2 * rb - 1) // (2 * rb)
    return pl.pallas_call(
        _repack_body,
        grid=(nb,),
        in_specs=[pl.BlockSpec((D, 2 * rb), lambda i: (0, i))],
        out_specs=pl.BlockSpec((rb, 128), lambda i: (i, 0)),
        out_shape=jax.ShapeDtypeStruct((nb * rb, 128), jnp.float32),
    )(tab_t)


def kernel(task, uni, task_table, uni_table):
    uni_v = uni.T.reshape(LI, 8, NW, 128).transpose(0, 2, 1, 3)
    task_v = task.T.reshape(LI, 8, NW, 128).transpose(0, 2, 1, 3)
    tab2 = _repack(uni_table.T, 4096)
    ttab2 = _repack(task_table.T, 500)
    out5 = _embed(uni_v, task_v, ttab2, tab2)
    return out5.transpose(2, 4, 0, 1, 3).reshape(B, L, D)
